# baseline (speedup 1.0000x reference)
import functools

import jax
import jax.numpy as jnp
from jax import lax
from jax.experimental import pallas as pl
from jax.experimental.pallas import tpu as pltpu

EPS = 1e-5
F32 = jnp.float32
BF16 = jnp.bfloat16
CH = 2


def _conv3x3_chunk(pad_ref, w_ref, n0, *, H, W, HWP):
    HW = H * W
    L = CH * HW
    wcol = lax.broadcasted_iota(jnp.int32, (1, L), 1) % W
    parts = []
    for dy in range(3):
        for dx in range(3):
            start = dy * W + dx
            taps = [pad_ref[:, n * HWP + start:n * HWP + start + HW]
                    for n in range(n0, n0 + CH)]
            tap = jnp.concatenate(taps, axis=1) if CH > 1 else taps[0]
            if dx == 0:
                tap = jnp.where(wcol >= 1, tap, jnp.zeros((), BF16))
            elif dx == 2:
                tap = jnp.where(wcol <= W - 2, tap, jnp.zeros((), BF16))
            parts.append(tap)
    stacked = jnp.concatenate(parts, axis=0)
    return jnp.dot(w_ref[...], stacked, preferred_element_type=F32)


def _conv_store_stats(pad_ref, w_ref, y_ref, s_ref, q_ref, *, NB, H, W, HWP):
    HW = H * W
    s = None
    q = None
    for n0 in range(0, NB, CH):
        y = _conv3x3_chunk(pad_ref, w_ref, n0, H=H, W=W, HWP=HWP)
        for i in range(CH):
            y_ref[n0 + i, :, :] = y[:, i * HW:(i + 1) * HW].astype(BF16)
        cs = jnp.sum(y, axis=1, keepdims=True)
        cq = jnp.sum(y * y, axis=1, keepdims=True)
        s = cs if s is None else s + cs
        q = cq if q is None else q + cq
    s_ref[0, :, :] = s
    q_ref[0, :, :] = q


def _conv_stats_kernel(x_ref, w_ref, y_ref, s_ref, q_ref, pad_ref,
                       *, NB, H, W, HWP):
    HW = H * W
    P = W + 1
    C = w_ref.shape[0]
    for n in range(NB):
        base = n * HWP
        pad_ref[:, base:base + P] = jnp.zeros((C, P), BF16)
        pad_ref[:, base + P + HW:base + HWP] = jnp.zeros((C, HWP - P - HW),
                                                         BF16)
        pad_ref[:, base + P:base + P + HW] = x_ref[n].astype(BF16)
    _conv_store_stats(pad_ref, w_ref, y_ref, s_ref, q_ref,
                      NB=NB, H=H, W=W, HWP=HWP)


def _bn_prelu_conv_stats_kernel(y1_ref, sc_ref, sh_ref, a_ref, w_ref,
                                y2_ref, s_ref, q_ref, pad_ref,
                                *, NB, H, W, HWP):
    HW = H * W
    P = W + 1
    C = w_ref.shape[0]
    a = a_ref[0]
    for n in range(NB):
        base = n * HWP
        pad_ref[:, base:base + P] = jnp.zeros((C, P), BF16)
        pad_ref[:, base + P + HW:base + HWP] = jnp.zeros((C, HWP - P - HW),
                                                         BF16)
        z = y1_ref[n].astype(F32) * sc_ref[...] + sh_ref[...]
        z = jnp.where(z >= 0.0, z, a * z)
        pad_ref[:, base + P:base + P + HW] = z.astype(BF16)
    _conv_store_stats(pad_ref, w_ref, y2_ref, s_ref, q_ref,
                      NB=NB, H=H, W=W, HWP=HWP)


def _bn_residual_kernel(x_ref, y2_ref, sc_ref, sh_ref, out_ref):
    out_ref[...] = x_ref[...] + (y2_ref[...].astype(F32) * sc_ref[...]
                                 + sh_ref[...])


def kernel(x, w1, b1, w2, b2, gamma1, beta1, gamma2, beta2, prelu_a):
    N, C, H, W = x.shape
    HW = H * W
    count = float(N * HW)

    NB = 8
    while N % NB:
        NB //= 2
    S = N // NB
    P = W + 1
    HWP = ((HW + 2 * P + 127) // 128) * 128

    x3 = x.reshape(N, C, HW)

    def pack_w(w):
        return jnp.transpose(w, (0, 2, 3, 1)).reshape(C, 9 * C).astype(BF16)

    w1p = pack_w(w1)
    w2p = pack_w(w2)
    g1 = gamma1.reshape(C, 1).astype(F32)
    be1 = beta1.reshape(C, 1).astype(F32)
    g2 = gamma2.reshape(C, 1).astype(F32)
    be2 = beta2.reshape(C, 1).astype(F32)
    a = prelu_a.reshape(1).astype(F32)

    act_in_spec = pl.BlockSpec((NB, C, HW), lambda n: (n, 0, 0))
    w_spec = pl.BlockSpec((C, 9 * C), lambda n: (0, 0))
    vec_spec = pl.BlockSpec((C, 1), lambda n: (0, 0))
    stat_spec = pl.BlockSpec((1, C, 1), lambda n: (n, 0, 0))
    smem_spec = pl.BlockSpec(memory_space=pltpu.MemorySpace.SMEM)
    pad_scratch = pltpu.VMEM((C, NB * HWP), BF16)
    cparams = pltpu.CompilerParams(dimension_semantics=("parallel",))

    bf_act_shape = jax.ShapeDtypeStruct((N, C, HW), BF16)
    stat_shape = jax.ShapeDtypeStruct((S, C, 1), F32)

    y1, s1, q1 = pl.pallas_call(
        functools.partial(_conv_stats_kernel, NB=NB, H=H, W=W, HWP=HWP),
        grid=(S,),
        in_specs=[act_in_spec, w_spec],
        out_specs=(act_in_spec, stat_spec, stat_spec),
        out_shape=(bf_act_shape, stat_shape, stat_shape),
        scratch_shapes=[pad_scratch],
        compiler_params=cparams,
    )(x3, w1p)

    def fold_bn(s, q, gamma, beta):
        mean = jnp.sum(s, axis=0) / count
        var = jnp.sum(q, axis=0) / count - mean * mean
        scale = gamma * lax.rsqrt(var + EPS)
        shift = beta - mean * scale
        return scale, shift

    scale1, shift1 = fold_bn(s1, q1, g1, be1)

    y2, s2, q2 = pl.pallas_call(
        functools.partial(_bn_prelu_conv_stats_kernel, NB=NB, H=H, W=W,
                          HWP=HWP),
        grid=(S,),
        in_specs=[act_in_spec, vec_spec, vec_spec, smem_spec, w_spec],
        out_specs=(act_in_spec, stat_spec, stat_spec),
        out_shape=(bf_act_shape, stat_shape, stat_shape),
        scratch_shapes=[pad_scratch],
        compiler_params=cparams,
    )(y1, scale1, shift1, a, w2p)

    scale2, shift2 = fold_bn(s2, q2, g2, be2)

    out = pl.pallas_call(
        _bn_residual_kernel,
        grid=(S,),
        in_specs=[act_in_spec, act_in_spec, vec_spec, vec_spec],
        out_specs=act_in_spec,
        out_shape=jax.ShapeDtypeStruct((N, C, HW), F32),
        compiler_params=cparams,
    )(x3, y2, scale2, shift2)

    return out.reshape(N, C, H, W)


# speedup vs baseline: 1.2087x; 1.2087x over previous
import functools

import jax
import jax.numpy as jnp
from jax import lax
from jax.experimental import pallas as pl
from jax.experimental.pallas import tpu as pltpu

EPS = 1e-5
F32 = jnp.float32
BF16 = jnp.bfloat16


def _fold_bn(s_ref, q_ref, g_ref, b_ref, count):
    mean = jnp.sum(s_ref[...], axis=0) / count
    var = jnp.sum(q_ref[...], axis=0) / count - mean * mean
    scale = g_ref[...] * lax.rsqrt(var + EPS)
    shift = b_ref[...] - mean * scale
    return scale, shift


def _conv3x3(pad_ref, w_ref, *, NB, H, W, HWP):
    HW = H * W
    L = NB * HW
    wcol = lax.broadcasted_iota(jnp.int32, (1, L), 1) % W
    parts = []
    for dy in range(3):
        for dx in range(3):
            start = dy * W + dx
            taps = [pad_ref[:, n * HWP + start:n * HWP + start + HW]
                    for n in range(NB)]
            tap = jnp.concatenate(taps, axis=1) if NB > 1 else taps[0]
            if dx == 0:
                tap = jnp.where(wcol >= 1, tap, jnp.zeros((), BF16))
            elif dx == 2:
                tap = jnp.where(wcol <= W - 2, tap, jnp.zeros((), BF16))
            parts.append(tap)
    stacked = jnp.concatenate(parts, axis=0)
    return jnp.dot(w_ref[...], stacked, preferred_element_type=F32)


def _conv_stats_kernel(x_ref, w_ref, y_ref, s_ref, q_ref, pad_ref,
                       *, NB, H, W, HWP):
    HW = H * W
    P = W + 1
    C = w_ref.shape[0]
    for n in range(NB):
        base = n * HWP
        pad_ref[:, base:base + P] = jnp.zeros((C, P), BF16)
        pad_ref[:, base + P + HW:base + HWP] = jnp.zeros((C, HWP - P - HW),
                                                         BF16)
        pad_ref[:, base + P:base + P + HW] = x_ref[n].astype(BF16)
    y = _conv3x3(pad_ref, w_ref, NB=NB, H=H, W=W, HWP=HWP)
    for n in range(NB):
        y_ref[n, :, :] = y[:, n * HW:(n + 1) * HW].astype(BF16)
    s_ref[0, :, :] = jnp.sum(y, axis=1, keepdims=True)
    q_ref[0, :, :] = jnp.sum(y * y, axis=1, keepdims=True)


def _bn_prelu_conv_stats_kernel(y1_ref, s1_ref, q1_ref, g1_ref, b1_ref,
                                a_ref, w_ref, y2_ref, s_ref, q_ref, pad_ref,
                                *, NB, H, W, HWP, count):
    HW = H * W
    P = W + 1
    C = w_ref.shape[0]
    a = a_ref[0]
    sc, sh = _fold_bn(s1_ref, q1_ref, g1_ref, b1_ref, count)
    for n in range(NB):
        base = n * HWP
        pad_ref[:, base:base + P] = jnp.zeros((C, P), BF16)
        pad_ref[:, base + P + HW:base + HWP] = jnp.zeros((C, HWP - P - HW),
                                                         BF16)
        z = y1_ref[n].astype(F32) * sc + sh
        z = jnp.where(z >= 0.0, z, a * z)
        pad_ref[:, base + P:base + P + HW] = z.astype(BF16)
    y = _conv3x3(pad_ref, w_ref, NB=NB, H=H, W=W, HWP=HWP)
    for n in range(NB):
        y2_ref[n, :, :] = y[:, n * HW:(n + 1) * HW].astype(BF16)
    s_ref[0, :, :] = jnp.sum(y, axis=1, keepdims=True)
    q_ref[0, :, :] = jnp.sum(y * y, axis=1, keepdims=True)


def _bn_residual_kernel(x_ref, y2_ref, s2_ref, q2_ref, g2_ref, b2_ref,
                        out_ref, *, count):
    sc, sh = _fold_bn(s2_ref, q2_ref, g2_ref, b2_ref, count)
    out_ref[...] = x_ref[...] + (y2_ref[...].astype(F32) * sc + sh)


def kernel(x, w1, b1, w2, b2, gamma1, beta1, gamma2, beta2, prelu_a):
    N, C, H, W = x.shape
    HW = H * W
    count = float(N * HW)

    NB = 8
    while N % NB:
        NB //= 2
    S = N // NB
    P = W + 1
    HWP = ((HW + 2 * P + 127) // 128) * 128

    x3 = x.reshape(N, C, HW)

    def pack_w(w):
        return jnp.transpose(w, (0, 2, 3, 1)).reshape(C, 9 * C).astype(BF16)

    w1p = pack_w(w1)
    w2p = pack_w(w2)
    g1 = gamma1.reshape(C, 1).astype(F32)
    be1 = beta1.reshape(C, 1).astype(F32)
    g2 = gamma2.reshape(C, 1).astype(F32)
    be2 = beta2.reshape(C, 1).astype(F32)
    a = prelu_a.reshape(1).astype(F32)

    act_spec = pl.BlockSpec((NB, C, HW), lambda n: (n, 0, 0))
    w_spec = pl.BlockSpec((C, 9 * C), lambda n: (0, 0))
    vec_spec = pl.BlockSpec((C, 1), lambda n: (0, 0))
    stat_spec = pl.BlockSpec((1, C, 1), lambda n: (n, 0, 0))
    stat_all_spec = pl.BlockSpec((S, C, 1), lambda n: (0, 0, 0))
    smem_spec = pl.BlockSpec(memory_space=pltpu.MemorySpace.SMEM)
    pad_scratch = pltpu.VMEM((C, NB * HWP), BF16)
    cparams = pltpu.CompilerParams(dimension_semantics=("parallel",))

    bf_act_shape = jax.ShapeDtypeStruct((N, C, HW), BF16)
    stat_shape = jax.ShapeDtypeStruct((S, C, 1), F32)

    y1, s1, q1 = pl.pallas_call(
        functools.partial(_conv_stats_kernel, NB=NB, H=H, W=W, HWP=HWP),
        grid=(S,),
        in_specs=[act_spec, w_spec],
        out_specs=(act_spec, stat_spec, stat_spec),
        out_shape=(bf_act_shape, stat_shape, stat_shape),
        scratch_shapes=[pad_scratch],
        compiler_params=cparams,
    )(x3, w1p)

    y2, s2, q2 = pl.pallas_call(
        functools.partial(_bn_prelu_conv_stats_kernel, NB=NB, H=H, W=W,
                          HWP=HWP, count=count),
        grid=(S,),
        in_specs=[act_spec, stat_all_spec, stat_all_spec, vec_spec, vec_spec,
                  smem_spec, w_spec],
        out_specs=(act_spec, stat_spec, stat_spec),
        out_shape=(bf_act_shape, stat_shape, stat_shape),
        scratch_shapes=[pad_scratch],
        compiler_params=cparams,
    )(y1, s1, q1, g1, be1, a, w2p)

    out = pl.pallas_call(
        functools.partial(_bn_residual_kernel, count=count),
        grid=(S,),
        in_specs=[act_spec, act_spec, stat_all_spec, stat_all_spec, vec_spec,
                  vec_spec],
        out_specs=act_spec,
        out_shape=jax.ShapeDtypeStruct((N, C, HW), F32),
        compiler_params=cparams,
    )(x3, y2, s2, q2, g2, be2)

    return out.reshape(N, C, H, W)


# speedup vs baseline: 1.2850x; 1.0631x over previous
import functools

import jax
import jax.numpy as jnp
from jax import lax
from jax.experimental import pallas as pl
from jax.experimental.pallas import tpu as pltpu

EPS = 1e-5
F32 = jnp.float32
BF16 = jnp.bfloat16


def _fold_bn(s_ref, q_ref, g_ref, b_ref, count):
    mean = jnp.sum(s_ref[...], axis=0) / count
    var = jnp.sum(q_ref[...], axis=0) / count - mean * mean
    scale = g_ref[...] * lax.rsqrt(var + EPS)
    shift = b_ref[...] - mean * scale
    return scale, shift


def _conv3x3(pad_ref, w_ref, *, NB, H, W, HWP):
    HW = H * W
    L = NB * HW
    PB = 128
    wcol = lax.broadcasted_iota(jnp.int32, (1, L), 1) % W
    parts = []
    for dy in range(3):
        for dx in range(3):
            start = PB + (dy - 1) * W + (dx - 1)
            taps = [pad_ref[:, n * HWP + start:n * HWP + start + HW]
                    for n in range(NB)]
            tap = jnp.concatenate(taps, axis=1) if NB > 1 else taps[0]
            if dx == 0:
                tap = jnp.where(wcol >= 1, tap, jnp.zeros((), BF16))
            elif dx == 2:
                tap = jnp.where(wcol <= W - 2, tap, jnp.zeros((), BF16))
            parts.append(tap)
    stacked = jnp.concatenate(parts, axis=0)
    return jnp.dot(w_ref[...], stacked, preferred_element_type=F32)


def _conv_stats_kernel(x_ref, w_ref, y_ref, s_ref, q_ref, pad_ref,
                       *, NB, H, W, HWP):
    HW = H * W
    PB = 128
    C = w_ref.shape[0]
    for n in range(NB):
        base = n * HWP
        pad_ref[:, base:base + PB] = jnp.zeros((C, PB), BF16)
        pad_ref[:, base + PB + HW:base + HWP] = jnp.zeros((C, HWP - PB - HW),
                                                          BF16)
        pad_ref[:, base + PB:base + PB + HW] = x_ref[n].astype(BF16)
    y = _conv3x3(pad_ref, w_ref, NB=NB, H=H, W=W, HWP=HWP)
    for n in range(NB):
        y_ref[n, :, :] = y[:, n * HW:(n + 1) * HW].astype(BF16)
    s_ref[0, :, :] = jnp.sum(y, axis=1, keepdims=True)
    q_ref[0, :, :] = jnp.sum(y * y, axis=1, keepdims=True)


def _bn_prelu_conv_stats_kernel(y1_ref, s1_ref, q1_ref, g1_ref, b1_ref,
                                a_ref, w_ref, y2_ref, s_ref, q_ref, pad_ref,
                                *, NB, H, W, HWP, count):
    HW = H * W
    PB = 128
    C = w_ref.shape[0]
    a = a_ref[0].astype(BF16)
    sc, sh = _fold_bn(s1_ref, q1_ref, g1_ref, b1_ref, count)
    scb = sc.astype(BF16)
    shb = sh.astype(BF16)
    for n in range(NB):
        base = n * HWP
        pad_ref[:, base:base + PB] = jnp.zeros((C, PB), BF16)
        pad_ref[:, base + PB + HW:base + HWP] = jnp.zeros((C, HWP - PB - HW),
                                                          BF16)
        z = y1_ref[n] * scb + shb
        z = jnp.where(z >= 0, z, a * z)
        pad_ref[:, base + PB:base + PB + HW] = z
    y = _conv3x3(pad_ref, w_ref, NB=NB, H=H, W=W, HWP=HWP)
    for n in range(NB):
        y2_ref[n, :, :] = y[:, n * HW:(n + 1) * HW].astype(BF16)
    s_ref[0, :, :] = jnp.sum(y, axis=1, keepdims=True)
    q_ref[0, :, :] = jnp.sum(y * y, axis=1, keepdims=True)


def _bn_residual_kernel(x_ref, y2_ref, s2_ref, q2_ref, g2_ref, b2_ref,
                        out_ref, *, count):
    sc, sh = _fold_bn(s2_ref, q2_ref, g2_ref, b2_ref, count)
    out_ref[...] = x_ref[...] + (y2_ref[...].astype(F32) * sc + sh)


def kernel(x, w1, b1, w2, b2, gamma1, beta1, gamma2, beta2, prelu_a):
    N, C, H, W = x.shape
    HW = H * W
    count = float(N * HW)

    NB = 8
    while N % NB:
        NB //= 2
    S = N // NB
    PB = 128
    HWP = ((HW + PB + W + 1 + 127) // 128) * 128

    x3 = x.reshape(N, C, HW)

    def pack_w(w):
        return jnp.transpose(w, (0, 2, 3, 1)).reshape(C, 9 * C).astype(BF16)

    w1p = pack_w(w1)
    w2p = pack_w(w2)
    g1 = gamma1.reshape(C, 1).astype(F32)
    be1 = beta1.reshape(C, 1).astype(F32)
    g2 = gamma2.reshape(C, 1).astype(F32)
    be2 = beta2.reshape(C, 1).astype(F32)
    a = prelu_a.reshape(1).astype(F32)

    act_spec = pl.BlockSpec((NB, C, HW), lambda n: (n, 0, 0))
    w_spec = pl.BlockSpec((C, 9 * C), lambda n: (0, 0))
    vec_spec = pl.BlockSpec((C, 1), lambda n: (0, 0))
    stat_spec = pl.BlockSpec((1, C, 1), lambda n: (n, 0, 0))
    stat_all_spec = pl.BlockSpec((S, C, 1), lambda n: (0, 0, 0))
    smem_spec = pl.BlockSpec(memory_space=pltpu.MemorySpace.SMEM)
    pad_scratch = pltpu.VMEM((C, NB * HWP), BF16)
    cparams = pltpu.CompilerParams(dimension_semantics=("parallel",))

    bf_act_shape = jax.ShapeDtypeStruct((N, C, HW), BF16)
    stat_shape = jax.ShapeDtypeStruct((S, C, 1), F32)

    y1, s1, q1 = pl.pallas_call(
        functools.partial(_conv_stats_kernel, NB=NB, H=H, W=W, HWP=HWP),
        grid=(S,),
        in_specs=[act_spec, w_spec],
        out_specs=(act_spec, stat_spec, stat_spec),
        out_shape=(bf_act_shape, stat_shape, stat_shape),
        scratch_shapes=[pad_scratch],
        compiler_params=cparams,
    )(x3, w1p)

    y2, s2, q2 = pl.pallas_call(
        functools.partial(_bn_prelu_conv_stats_kernel, NB=NB, H=H, W=W,
                          HWP=HWP, count=count),
        grid=(S,),
        in_specs=[act_spec, stat_all_spec, stat_all_spec, vec_spec, vec_spec,
                  smem_spec, w_spec],
        out_specs=(act_spec, stat_spec, stat_spec),
        out_shape=(bf_act_shape, stat_shape, stat_shape),
        scratch_shapes=[pad_scratch],
        compiler_params=cparams,
    )(y1, s1, q1, g1, be1, a, w2p)

    out = pl.pallas_call(
        functools.partial(_bn_residual_kernel, count=count),
        grid=(S,),
        in_specs=[act_spec, act_spec, stat_all_spec, stat_all_spec, vec_spec,
                  vec_spec],
        out_specs=act_spec,
        out_shape=jax.ShapeDtypeStruct((N, C, HW), F32),
        compiler_params=cparams,
    )(x3, y2, s2, q2, g2, be2)

    return out.reshape(N, C, H, W)


# speedup vs baseline: 1.3299x; 1.0350x over previous
import functools

import jax
import jax.numpy as jnp
from jax import lax
from jax.experimental import pallas as pl
from jax.experimental.pallas import tpu as pltpu

EPS = 1e-5
F32 = jnp.float32
BF16 = jnp.bfloat16


def _fold_bn(s_ref, q_ref, g_ref, b_ref, count):
    mean = jnp.sum(s_ref[...], axis=0) / count
    var = jnp.sum(q_ref[...], axis=0) / count - mean * mean
    scale = g_ref[...] * lax.rsqrt(var + EPS)
    shift = b_ref[...] - mean * scale
    return scale, shift


def _conv3x3(pad_ref, w_ref, *, NB, H, W, HWP):
    HW = H * W
    L = NB * HW
    PB = 128
    wcol = lax.broadcasted_iota(jnp.int32, (1, L), 1) % W
    parts = []
    for dy in range(3):
        for dx in range(3):
            start = PB + (dy - 1) * W + (dx - 1)
            taps = [pad_ref[:, n * HWP + start:n * HWP + start + HW]
                    for n in range(NB)]
            tap = jnp.concatenate(taps, axis=1) if NB > 1 else taps[0]
            if dx == 0:
                tap = jnp.where(wcol >= 1, tap, jnp.zeros((), BF16))
            elif dx == 2:
                tap = jnp.where(wcol <= W - 2, tap, jnp.zeros((), BF16))
            parts.append(tap)
    stacked = jnp.concatenate(parts, axis=0)
    return jnp.dot(w_ref[...], stacked, preferred_element_type=F32)


def _conv_stats_kernel(x_ref, w_ref, y_ref, xb_ref, s_ref, q_ref, pad_ref,
                       *, NB, H, W, HWP):
    HW = H * W
    PB = 128
    C = w_ref.shape[0]
    for n in range(NB):
        base = n * HWP
        pad_ref[:, base:base + PB] = jnp.zeros((C, PB), BF16)
        pad_ref[:, base + PB + HW:base + HWP] = jnp.zeros((C, HWP - PB - HW),
                                                          BF16)
        xb = x_ref[n].astype(BF16)
        xb_ref[n, :, :] = xb
        pad_ref[:, base + PB:base + PB + HW] = xb
    y = _conv3x3(pad_ref, w_ref, NB=NB, H=H, W=W, HWP=HWP)
    for n in range(NB):
        y_ref[n, :, :] = y[:, n * HW:(n + 1) * HW].astype(BF16)
    s_ref[0, :, :] = jnp.sum(y, axis=1, keepdims=True)
    q_ref[0, :, :] = jnp.sum(y * y, axis=1, keepdims=True)


def _bn_prelu_conv_stats_kernel(y1_ref, s1_ref, q1_ref, g1_ref, b1_ref,
                                a_ref, w_ref, y2_ref, s_ref, q_ref, pad_ref,
                                *, NB, H, W, HWP, count):
    HW = H * W
    PB = 128
    C = w_ref.shape[0]
    a = a_ref[0].astype(BF16)
    sc, sh = _fold_bn(s1_ref, q1_ref, g1_ref, b1_ref, count)
    scb = sc.astype(BF16)
    shb = sh.astype(BF16)
    for n in range(NB):
        base = n * HWP
        pad_ref[:, base:base + PB] = jnp.zeros((C, PB), BF16)
        pad_ref[:, base + PB + HW:base + HWP] = jnp.zeros((C, HWP - PB - HW),
                                                          BF16)
        z = y1_ref[n] * scb + shb
        z = jnp.where(z >= 0, z, a * z)
        pad_ref[:, base + PB:base + PB + HW] = z
    y = _conv3x3(pad_ref, w_ref, NB=NB, H=H, W=W, HWP=HWP)
    for n in range(NB):
        y2_ref[n, :, :] = y[:, n * HW:(n + 1) * HW].astype(BF16)
    s_ref[0, :, :] = jnp.sum(y, axis=1, keepdims=True)
    q_ref[0, :, :] = jnp.sum(y * y, axis=1, keepdims=True)


def _bn_residual_kernel(xb_ref, y2_ref, s2_ref, q2_ref, g2_ref, b2_ref,
                        out_ref, *, count):
    sc, sh = _fold_bn(s2_ref, q2_ref, g2_ref, b2_ref, count)
    out_ref[...] = xb_ref[...].astype(F32) + (y2_ref[...].astype(F32) * sc
                                              + sh)


def kernel(x, w1, b1, w2, b2, gamma1, beta1, gamma2, beta2, prelu_a):
    N, C, H, W = x.shape
    HW = H * W
    count = float(N * HW)

    NB = 8
    while N % NB:
        NB //= 2
    S = N // NB
    PB = 128
    HWP = ((HW + PB + W + 1 + 127) // 128) * 128

    x3 = x.reshape(N, C, HW)

    def pack_w(w):
        return jnp.transpose(w, (0, 2, 3, 1)).reshape(C, 9 * C).astype(BF16)

    w1p = pack_w(w1)
    w2p = pack_w(w2)
    g1 = gamma1.reshape(C, 1).astype(F32)
    be1 = beta1.reshape(C, 1).astype(F32)
    g2 = gamma2.reshape(C, 1).astype(F32)
    be2 = beta2.reshape(C, 1).astype(F32)
    a = prelu_a.reshape(1).astype(F32)

    act_spec = pl.BlockSpec((NB, C, HW), lambda n: (n, 0, 0))
    w_spec = pl.BlockSpec((C, 9 * C), lambda n: (0, 0))
    vec_spec = pl.BlockSpec((C, 1), lambda n: (0, 0))
    stat_spec = pl.BlockSpec((1, C, 1), lambda n: (n, 0, 0))
    stat_all_spec = pl.BlockSpec((S, C, 1), lambda n: (0, 0, 0))
    smem_spec = pl.BlockSpec(memory_space=pltpu.MemorySpace.SMEM)
    pad_scratch = pltpu.VMEM((C, NB * HWP), BF16)
    cparams = pltpu.CompilerParams(dimension_semantics=("parallel",))

    bf_act_shape = jax.ShapeDtypeStruct((N, C, HW), BF16)
    stat_shape = jax.ShapeDtypeStruct((S, C, 1), F32)

    y1, xb, s1, q1 = pl.pallas_call(
        functools.partial(_conv_stats_kernel, NB=NB, H=H, W=W, HWP=HWP),
        grid=(S,),
        in_specs=[act_spec, w_spec],
        out_specs=(act_spec, act_spec, stat_spec, stat_spec),
        out_shape=(bf_act_shape, bf_act_shape, stat_shape, stat_shape),
        scratch_shapes=[pad_scratch],
        compiler_params=cparams,
    )(x3, w1p)

    y2, s2, q2 = pl.pallas_call(
        functools.partial(_bn_prelu_conv_stats_kernel, NB=NB, H=H, W=W,
                          HWP=HWP, count=count),
        grid=(S,),
        in_specs=[act_spec, stat_all_spec, stat_all_spec, vec_spec, vec_spec,
                  smem_spec, w_spec],
        out_specs=(act_spec, stat_spec, stat_spec),
        out_shape=(bf_act_shape, stat_shape, stat_shape),
        scratch_shapes=[pad_scratch],
        compiler_params=cparams,
    )(y1, s1, q1, g1, be1, a, w2p)

    out = pl.pallas_call(
        functools.partial(_bn_residual_kernel, count=count),
        grid=(S,),
        in_specs=[act_spec, act_spec, stat_all_spec, stat_all_spec, vec_spec,
                  vec_spec],
        out_specs=act_spec,
        out_shape=jax.ShapeDtypeStruct((N, C, HW), F32),
        compiler_params=cparams,
    )(xb, y2, s2, q2, g2, be2)

    return out.reshape(N, C, H, W)


# speedup vs baseline: 1.3316x; 1.0012x over previous
import functools

import jax
import jax.numpy as jnp
from jax import lax
from jax.experimental import pallas as pl
from jax.experimental.pallas import tpu as pltpu

EPS = 1e-5
F32 = jnp.float32
BF16 = jnp.bfloat16


def _fold_bn(s_ref, q_ref, g_ref, b_ref, count):
    mean = jnp.sum(s_ref[...], axis=0) / count
    var = jnp.sum(q_ref[...], axis=0) / count - mean * mean
    scale = g_ref[...] * lax.rsqrt(var + EPS)
    shift = b_ref[...] - mean * scale
    return scale, shift


def _conv3x3(pad_ref, w_ref, *, NB, H, W, HWP):
    HW = H * W
    L = NB * HW
    PB = 128
    wcol = lax.broadcasted_iota(jnp.int32, (1, L), 1) % W
    parts = []
    for dy in range(3):
        for dx in range(3):
            start = PB + (dy - 1) * W + (dx - 1)
            taps = [pad_ref[:, n * HWP + start:n * HWP + start + HW]
                    for n in range(NB)]
            tap = jnp.concatenate(taps, axis=1) if NB > 1 else taps[0]
            if dx == 0:
                tap = jnp.where(wcol >= 1, tap, jnp.zeros((), BF16))
            elif dx == 2:
                tap = jnp.where(wcol <= W - 2, tap, jnp.zeros((), BF16))
            parts.append(tap)
    stacked = jnp.concatenate(parts, axis=0)
    return jnp.dot(w_ref[...], stacked, preferred_element_type=F32)


def _conv_stats_kernel(x_ref, w_ref, y_ref, xb_ref, s_ref, q_ref, pad_ref,
                       *, NB, H, W, HWP):
    HW = H * W
    PB = 128
    C = w_ref.shape[0]
    for n in range(NB):
        base = n * HWP
        pad_ref[:, base:base + PB] = jnp.zeros((C, PB), BF16)
        pad_ref[:, base + PB + HW:base + HWP] = jnp.zeros((C, HWP - PB - HW),
                                                          BF16)
        xb = x_ref[n].astype(BF16)
        xb_ref[n, :, :] = xb
        pad_ref[:, base + PB:base + PB + HW] = xb
    y = _conv3x3(pad_ref, w_ref, NB=NB, H=H, W=W, HWP=HWP)
    for n in range(NB):
        y_ref[n, :, :] = y[:, n * HW:(n + 1) * HW].astype(BF16)
    s_ref[0, :, :] = jnp.sum(y, axis=1, keepdims=True)
    q_ref[0, :, :] = jnp.sum(y * y, axis=1, keepdims=True)


def _bn_prelu_conv_stats_kernel(y1_ref, s1_ref, q1_ref, g1_ref, b1_ref,
                                a_ref, w_ref, y2_ref, s_ref, q_ref, pad_ref,
                                *, NB, H, W, HWP, count):
    HW = H * W
    PB = 128
    C = w_ref.shape[0]
    a = a_ref[0].astype(BF16)
    sc, sh = _fold_bn(s1_ref, q1_ref, g1_ref, b1_ref, count)
    scb = sc.astype(BF16)
    shb = sh.astype(BF16)
    for n in range(NB):
        base = n * HWP
        pad_ref[:, base:base + PB] = jnp.zeros((C, PB), BF16)
        pad_ref[:, base + PB + HW:base + HWP] = jnp.zeros((C, HWP - PB - HW),
                                                          BF16)
        z = y1_ref[n] * scb + shb
        z = jnp.where(z >= 0, z, a * z)
        pad_ref[:, base + PB:base + PB + HW] = z
    y = _conv3x3(pad_ref, w_ref, NB=NB, H=H, W=W, HWP=HWP)
    for n in range(NB):
        y2_ref[n, :, :] = y[:, n * HW:(n + 1) * HW].astype(BF16)
    s_ref[0, :, :] = jnp.sum(y, axis=1, keepdims=True)
    q_ref[0, :, :] = jnp.sum(y * y, axis=1, keepdims=True)


def _bn_residual_kernel(xb_ref, y2_ref, s2_ref, q2_ref, g2_ref, b2_ref,
                        out_ref, *, count):
    sc, sh = _fold_bn(s2_ref, q2_ref, g2_ref, b2_ref, count)
    out_ref[...] = xb_ref[...].astype(F32) + (y2_ref[...].astype(F32) * sc
                                              + sh)


def kernel(x, w1, b1, w2, b2, gamma1, beta1, gamma2, beta2, prelu_a):
    N, C, H, W = x.shape
    HW = H * W
    count = float(N * HW)

    NB = 8
    while N % NB:
        NB //= 2
    S = N // NB
    PB = 128
    HWP = ((HW + PB + W + 1 + 127) // 128) * 128

    x3 = x.reshape(N, C, HW)

    def pack_w(w):
        return jnp.transpose(w, (0, 2, 3, 1)).reshape(C, 9 * C).astype(BF16)

    w1p = pack_w(w1)
    w2p = pack_w(w2)
    g1 = gamma1.reshape(C, 1).astype(F32)
    be1 = beta1.reshape(C, 1).astype(F32)
    g2 = gamma2.reshape(C, 1).astype(F32)
    be2 = beta2.reshape(C, 1).astype(F32)
    a = prelu_a.reshape(1).astype(F32)

    act_spec = pl.BlockSpec((NB, C, HW), lambda n: (n, 0, 0))
    w_spec = pl.BlockSpec((C, 9 * C), lambda n: (0, 0))
    vec_spec = pl.BlockSpec((C, 1), lambda n: (0, 0))
    stat_spec = pl.BlockSpec((1, C, 1), lambda n: (n, 0, 0))
    stat_all_spec = pl.BlockSpec((S, C, 1), lambda n: (0, 0, 0))
    smem_spec = pl.BlockSpec(memory_space=pltpu.MemorySpace.SMEM)
    pad_scratch = pltpu.VMEM((C, NB * HWP), BF16)
    cparams = pltpu.CompilerParams(dimension_semantics=("parallel",))

    bf_act_shape = jax.ShapeDtypeStruct((N, C, HW), BF16)
    stat_shape = jax.ShapeDtypeStruct((S, C, 1), F32)

    y1, xb, s1, q1 = pl.pallas_call(
        functools.partial(_conv_stats_kernel, NB=NB, H=H, W=W, HWP=HWP),
        grid=(S,),
        in_specs=[act_spec, w_spec],
        out_specs=(act_spec, act_spec, stat_spec, stat_spec),
        out_shape=(bf_act_shape, bf_act_shape, stat_shape, stat_shape),
        scratch_shapes=[pad_scratch],
        compiler_params=cparams,
    )(x3, w1p)

    y2, s2, q2 = pl.pallas_call(
        functools.partial(_bn_prelu_conv_stats_kernel, NB=NB, H=H, W=W,
                          HWP=HWP, count=count),
        grid=(S,),
        in_specs=[act_spec, stat_all_spec, stat_all_spec, vec_spec, vec_spec,
                  smem_spec, w_spec],
        out_specs=(act_spec, stat_spec, stat_spec),
        out_shape=(bf_act_shape, stat_shape, stat_shape),
        scratch_shapes=[pad_scratch],
        compiler_params=cparams,
    )(y1, s1, q1, g1, be1, a, w2p)

    NB3 = 2 * NB if N % (2 * NB) == 0 else NB
    act3_spec = pl.BlockSpec((NB3, C, HW), lambda n: (n, 0, 0))
    out = pl.pallas_call(
        functools.partial(_bn_residual_kernel, count=count),
        grid=(N // NB3,),
        in_specs=[act3_spec, act3_spec, stat_all_spec, stat_all_spec,
                  vec_spec, vec_spec],
        out_specs=act3_spec,
        out_shape=jax.ShapeDtypeStruct((N, C, HW), F32),
        compiler_params=cparams,
    )(xb, y2, s2, q2, g2, be2)

    return out.reshape(N, C, H, W)
